# baseline (speedup 1.0000x reference)
import math

import jax
import jax.numpy as jnp
from jax import lax
from jax.experimental import pallas as pl
from jax.experimental.pallas import tpu as pltpu
from jax.experimental.pallas import tpu_sc as plsc

_VOCAB = 1000000
_DIM = 64
_B = 4096
_S = 200
_NW = 32
_BW = _B // _NW
_TB = _B // 128
_SCALE = 1.0 / math.sqrt(_DIM)
_L = 16
_NBUF = 4
_PITCH = 129


def _embed_kernel(idst_hbm, table_hbm, out_hbm,
                  idx_v, gbufs, obufs, gsems, ssems):
    wid = lax.axis_index("s") * 2 + lax.axis_index("c")

    pltpu.sync_copy(idst_hbm.at[:, pl.ds(wid * _BW, _BW)], idx_v)

    iota = lax.iota(jnp.int32, _L)
    dvecs = [iota + c * _L for c in range(_DIM // _L)]

    def start_gather(s, b):
        pltpu.async_copy(table_hbm.at[idx_v.at[s]], gbufs[b], gsems[b])

    def transpose_scale(b):
        gbuf, obuf = gbufs[b], obufs[b]

        @plsc.parallel_loop(0, _BW, unroll=4)
        def _(bm):
            bmv = iota * 0 + bm
            for c in range(_DIM // _L):
                v = gbuf[bm, pl.ds(c * _L, _L)] * _SCALE
                plsc.store_scatter(obuf, [dvecs[c], bmv], v)

    def store_out(s, b):
        for td in range(8):
            pltpu.async_copy(
                obufs[b].at[pl.ds(td * 8, 8), pl.ds(0, 128)],
                out_hbm.at[s, td, wid], ssems[b])

    def wait_stores(s, b):
        for td in range(8):
            pltpu.make_async_copy(
                obufs[b].at[pl.ds(td * 8, 8), pl.ds(0, 128)],
                out_hbm.at[s, td, wid], ssems[b]).wait()

    def step(s, b, wait_store, more_gathers):
        pltpu.make_async_copy(table_hbm.at[idx_v.at[s]], gbufs[b],
                              gsems[b]).wait()
        if wait_store:
            wait_stores(s, b)
        transpose_scale(b)
        if more_gathers:
            start_gather(s + _NBUF, b)
        store_out(s, b)

    for b in range(_NBUF):
        start_gather(b, b)
    for b in range(_NBUF):
        step(b, b, wait_store=False, more_gathers=True)

    def loop_body(ss, _):
        s = ss * _NBUF
        for b in range(_NBUF):
            step(s + b, b, wait_store=True, more_gathers=True)
        return 0
    lax.fori_loop(1, _S // _NBUF - 1, loop_body, 0)

    for b in range(_NBUF):
        step(_S - _NBUF + b, b, wait_store=True, more_gathers=False)
    for b in range(_NBUF):
        wait_stores(0, b)


_NFULL = _VOCAB // 128
_PER_W1 = _NFULL // _NW


def _transpose_kernel(wt_hbm, wtail_hbm, wlin_hbm, wblks, wpad, lbufs,
                      isems, osems):
    wid = lax.axis_index("s") * 2 + lax.axis_index("c")
    iota = lax.iota(jnp.int32, _L)
    dv129 = [(iota + c * _L) * _PITCH for c in range(_DIM // _L)]

    def blk_col(k):
        return (wid * _PER_W1 + k) * 128

    def start_load(k, b):
        pltpu.async_copy(wt_hbm.at[:, pl.ds(blk_col(k), 128)], wblks[b],
                         isems[b])

    dvecs = [iota + c * _L for c in range(_DIM // _L)]

    def transpose_block(b):
        @plsc.parallel_loop(0, _DIM, unroll=4)
        def _(d):
            for g in range(128 // _L):
                wpad[d, pl.ds(g * _L, _L)] = wblks[b][d, pl.ds(g * _L, _L)]

        @plsc.parallel_loop(0, 128, unroll=4)
        def _(v):
            vv = iota * 0 + v
            for c in range(_DIM // _L):
                row = plsc.load_gather(wpad, [dvecs[c], vv])
                lbufs[b][v, pl.ds(c * _L, _L)] = row

    def store_block(k, b):
        pltpu.async_copy(lbufs[b], wlin_hbm.at[pl.ds(blk_col(k), 128)],
                         osems[b])

    def wait_store(b):
        pltpu.make_async_copy(lbufs[b], wlin_hbm.at[pl.ds(0, 128)],
                              osems[b]).wait()

    def step(k, b, wait_st, more):
        pltpu.make_async_copy(wt_hbm.at[:, pl.ds(blk_col(k), 128)],
                              wblks[b], isems[b]).wait()
        if wait_st:
            wait_store(b)
        transpose_block(b)
        if more:
            start_load(k + 2, b)
        store_block(k, b)

    for b in range(2):
        start_load(b, b)
    for b in range(2):
        step(b, b, wait_st=False, more=True)

    def loop_body(kk, _):
        k = kk * 2
        for b in range(2):
            step(k + b, b, wait_st=True, more=True)
        return 0
    lax.fori_loop(1, _PER_W1 // 2 - 1, loop_body, 0)

    for b in range(2):
        step(_PER_W1 - 2 + b, b, wait_st=True, more=False)
    for b in range(2):
        wait_store(b)

    @pl.when(wid < 4)
    def _():
        col = (_NFULL - 4 + wid) * 128
        pltpu.async_copy(wt_hbm.at[:, pl.ds(col, 128)], wblks[0],
                         isems[0]).wait()
        transpose_block(0)
        pltpu.async_copy(lbufs[0], wlin_hbm.at[pl.ds(col, 128)],
                         osems[0]).wait()

    @pl.when(wid == 4)
    def _():
        pltpu.async_copy(wtail_hbm, wblks[0], isems[0]).wait()
        transpose_block(0)
        pltpu.async_copy(lbufs[0], wlin_hbm.at[pl.ds(_VOCAB - 128, 128)],
                         osems[0]).wait()


@jax.jit
def _transpose(wt, wtail):
    mesh = plsc.VectorSubcoreMesh(core_axis_name="c", subcore_axis_name="s")
    return pl.kernel(
        _transpose_kernel,
        mesh=mesh,
        out_type=jax.ShapeDtypeStruct((_VOCAB, 128), jnp.float32),
        scratch_types=[
            [pltpu.VMEM((_DIM, 128), jnp.float32) for _ in range(2)],
            pltpu.VMEM((_DIM, _PITCH), jnp.float32),
            [pltpu.VMEM((128, 128), jnp.float32) for _ in range(2)],
            [pltpu.SemaphoreType.DMA for _ in range(2)],
            [pltpu.SemaphoreType.DMA for _ in range(2)],
        ],
        compiler_params=pltpu.CompilerParams(use_tc_tiling_on_sc=True,
                                             needs_layout_passes=False),
    )(wt, wtail)


@jax.jit
def _embed(ids_t, W):
    mesh = plsc.VectorSubcoreMesh(core_axis_name="c", subcore_axis_name="s")
    return pl.kernel(
        _embed_kernel,
        mesh=mesh,
        out_type=jax.ShapeDtypeStruct((_S, 8, _TB, 8, 128), jnp.float32),
        scratch_types=[
            pltpu.VMEM((_S, _BW), jnp.int32),
            [pltpu.VMEM((_BW, 128), jnp.float32) for _ in range(_NBUF)],
            [pltpu.VMEM((_DIM, _PITCH), jnp.float32) for _ in range(_NBUF)],
            [pltpu.SemaphoreType.DMA for _ in range(_NBUF)],
            [pltpu.SemaphoreType.DMA for _ in range(_NBUF)],
        ],
        compiler_params=pltpu.CompilerParams(use_tc_tiling_on_sc=False,
                                             needs_layout_passes=False),
    )(ids_t, W)


def kernel(ids, W):
    wt = W.T
    wtail = wt[:, _VOCAB - 128:]
    Wp = _transpose(wt, wtail)
    o5 = _embed(ids.astype(jnp.int32).T, Wp)
    return o5.transpose(2, 4, 0, 1, 3).reshape(_B, _S, _DIM)


# speedup vs baseline: 1.4485x; 1.4485x over previous
import math

import jax
import jax.numpy as jnp
from jax import lax
from jax.experimental import pallas as pl
from jax.experimental.pallas import tpu as pltpu
from jax.experimental.pallas import tpu_sc as plsc

_VOCAB = 1000000
_DIM = 64
_B = 4096
_S = 200
_NW = 32
_BW = _B // _NW
_TB = _B // 128
_SCALE = 1.0 / math.sqrt(_DIM)
_L = 16
_NBUF = 4
_PITCH = 129


def _embed_kernel(idst_hbm, table_hbm, out_hbm,
                  idx_v, gbufs, obufs, gsems, ssems):
    wid = lax.axis_index("s") * 2 + lax.axis_index("c")

    pltpu.sync_copy(idst_hbm.at[:, pl.ds(wid * _BW, _BW)], idx_v)

    iota = lax.iota(jnp.int32, _L)
    dvecs = [iota + c * _L for c in range(_DIM // _L)]

    def start_gather(s, b):
        pltpu.async_copy(table_hbm.at[idx_v.at[s]], gbufs[b], gsems[b])

    def transpose_scale(b):
        gbuf, obuf = gbufs[b], obufs[b]

        @plsc.parallel_loop(0, _BW, unroll=4)
        def _(bm):
            bmv = iota * 0 + bm
            for c in range(_DIM // _L):
                v = gbuf[bm, pl.ds(c * _L, _L)] * _SCALE
                plsc.store_scatter(obuf, [dvecs[c], bmv], v)

    def store_out(s, b):
        for td in range(8):
            pltpu.async_copy(
                obufs[b].at[pl.ds(td * 8, 8), pl.ds(0, 128)],
                out_hbm.at[s, td, wid], ssems[b])

    def wait_stores(s, b):
        for td in range(8):
            pltpu.make_async_copy(
                obufs[b].at[pl.ds(td * 8, 8), pl.ds(0, 128)],
                out_hbm.at[s, td, wid], ssems[b]).wait()

    def step(s, b, wait_store, more_gathers):
        pltpu.make_async_copy(table_hbm.at[idx_v.at[s]], gbufs[b],
                              gsems[b]).wait()
        if wait_store:
            wait_stores(s, b)
        transpose_scale(b)
        if more_gathers:
            start_gather(s + _NBUF, b)
        store_out(s, b)

    for b in range(_NBUF):
        start_gather(b, b)
    for b in range(_NBUF):
        step(b, b, wait_store=False, more_gathers=True)

    def loop_body(ss, _):
        s = ss * _NBUF
        for b in range(_NBUF):
            step(s + b, b, wait_store=True, more_gathers=True)
        return 0
    lax.fori_loop(1, _S // _NBUF - 1, loop_body, 0)

    for b in range(_NBUF):
        step(_S - _NBUF + b, b, wait_store=True, more_gathers=False)
    for b in range(_NBUF):
        wait_stores(0, b)


@jax.jit
def _embed(ids_t, W):
    mesh = plsc.VectorSubcoreMesh(core_axis_name="c", subcore_axis_name="s")
    return pl.kernel(
        _embed_kernel,
        mesh=mesh,
        out_type=jax.ShapeDtypeStruct((_S, 8, _TB, 8, 128), jnp.float32),
        scratch_types=[
            pltpu.VMEM((_S, _BW), jnp.int32),
            [pltpu.VMEM((_BW, 128), jnp.float32) for _ in range(_NBUF)],
            [pltpu.VMEM((_DIM, _PITCH), jnp.float32) for _ in range(_NBUF)],
            [pltpu.SemaphoreType.DMA for _ in range(_NBUF)],
            [pltpu.SemaphoreType.DMA for _ in range(_NBUF)],
        ],
        compiler_params=pltpu.CompilerParams(use_tc_tiling_on_sc=False,
                                             needs_layout_passes=False),
    )(ids_t, W)


def kernel(ids, W):
    Wp = jnp.pad(W, ((0, 0), (0, 128 - _DIM)))
    o5 = _embed(ids.astype(jnp.int32).T, Wp)
    return o5.transpose(2, 4, 0, 1, 3).reshape(_B, _S, _DIM)
